# baseline (speedup 1.0000x reference)
import jax
import jax.numpy as jnp
from jax.experimental import pallas as pl
from jax.experimental.pallas import tpu as pltpu

_BLOCK_S = 2560


def _add_kernel(x_ref, pos_ref, out_ref):
    out_ref[0] = x_ref[0] + pos_ref[...]


def kernel(x, pos_table):
    batch, seq_len, d_model = x.shape
    bs = _BLOCK_S
    grid = (pl.cdiv(seq_len, bs), batch)
    return pl.pallas_call(
        _add_kernel,
        grid=grid,
        in_specs=[
            pl.BlockSpec((1, bs, d_model), lambda i, j: (j, i, 0)),
            pl.BlockSpec((bs, d_model), lambda i, j: (i, 0)),
        ],
        out_specs=pl.BlockSpec((1, bs, d_model), lambda i, j: (j, i, 0)),
        out_shape=jax.ShapeDtypeStruct(x.shape, x.dtype),
        compiler_params=pltpu.CompilerParams(
            dimension_semantics=("parallel", "parallel"),
            vmem_limit_bytes=63 * 1024 * 1024,
        ),
    )(x, pos_table)


# speedup vs baseline: 1.0448x; 1.0448x over previous
import jax
import jax.numpy as jnp
from jax.experimental import pallas as pl
from jax.experimental.pallas import tpu as pltpu

_BLOCK_S = 2048


def _add_kernel(x_ref, pos_ref, out_ref):
    out_ref[0] = x_ref[0] + pos_ref[...]


def kernel(x, pos_table):
    batch, seq_len, d_model = x.shape
    bs = _BLOCK_S
    grid = (seq_len // bs, batch)
    return pl.pallas_call(
        _add_kernel,
        grid=grid,
        in_specs=[
            pl.BlockSpec((1, bs, d_model), lambda i, j: (j, i, 0)),
            pl.BlockSpec((bs, d_model), lambda i, j: (i, 0)),
        ],
        out_specs=pl.BlockSpec((1, bs, d_model), lambda i, j: (j, i, 0)),
        out_shape=jax.ShapeDtypeStruct(x.shape, x.dtype),
        compiler_params=pltpu.CompilerParams(
            dimension_semantics=("parallel", "parallel"),
        ),
    )(x, pos_table)
